# Initial kernel scaffold; baseline (speedup 1.0000x reference)
import jax
import jax.numpy as jnp
from jax.experimental import pallas as pl


def kernel(sketchs, sketch_lengths, table):
    raise NotImplementedError("write your pallas kernel here")


# speedup vs baseline: 123.1433x; 123.1433x over previous
import functools

import jax
import jax.numpy as jnp
import numpy as np
from jax import lax
from jax.experimental import pallas as pl
from jax.experimental.pallas import tpu as pltpu
from jax.experimental.pallas import tpu_sc as plsc

B = 16384
L = 200
EMB = 128
VPAD = 16
NW = L // 8
NWPAD = 32

NUM_WORKERS = 32
ROWS_PER_W = B // NUM_WORKERS
GROUPS = ROWS_PER_W // 16


def _shift_matrices():
    s_lo = np.zeros((L, NWPAD), np.float32)
    s_hi = np.zeros((L, NWPAD), np.float32)
    for l in range(L):
        p, k = divmod(l, 8)
        if k < 4:
            s_lo[l, p] = float(16 ** k)
        else:
            s_hi[l, p] = float(16 ** (k - 4))
    return jnp.asarray(s_lo), jnp.asarray(s_hi)


def _tc_pack(sketchs, lengths_col, s_lo, s_hi):
    BM = 2048

    def pack(x_ref, len_ref, slo_ref, shi_ref, out_ref):
        pos = lax.broadcasted_iota(jnp.int32, (BM, L), 1)
        valid = pos < len_ref[...]
        xm = jnp.where(valid, x_ref[...], 15).astype(jnp.float32)
        y_lo = jnp.dot(xm, slo_ref[...], precision=lax.Precision.HIGHEST,
                       preferred_element_type=jnp.float32)
        y_hi = jnp.dot(xm, shi_ref[...], precision=lax.Precision.HIGHEST,
                       preferred_element_type=jnp.float32)
        out_ref[...] = (y_lo + 0.5).astype(jnp.int32) + (
            (y_hi + 0.5).astype(jnp.int32) << 16
        )

    return pl.pallas_call(
        pack,
        grid=(B // BM,),
        in_specs=[
            pl.BlockSpec((BM, L), lambda i: (i, 0)),
            pl.BlockSpec((BM, 1), lambda i: (i, 0)),
            pl.BlockSpec((L, NWPAD), lambda i: (0, 0)),
            pl.BlockSpec((L, NWPAD), lambda i: (0, 0)),
        ],
        out_specs=pl.BlockSpec((BM, NWPAD), lambda i: (i, 0)),
        out_shape=jax.ShapeDtypeStruct((B, NWPAD), jnp.int32),
    )(sketchs, lengths_col, s_lo, s_hi)


def _sc_histogram(packed):
    mesh = plsc.VectorSubcoreMesh(core_axis_name="c", subcore_axis_name="s")

    @functools.partial(
        pl.kernel,
        mesh=mesh,
        out_type=jax.ShapeDtypeStruct((B * VPAD,), jnp.float32),
        compiler_params=pltpu.CompilerParams(
            use_tc_tiling_on_sc=False, needs_layout_passes=False
        ),
        scratch_types=[
            pltpu.VMEM((ROWS_PER_W, NWPAD), jnp.int32),
            pltpu.VMEM((ROWS_PER_W * VPAD,), jnp.float32),
        ],
    )
    def hist(packed_hbm, counts_hbm, words_v, counts_v):
        wid = lax.axis_index("s") * 2 + lax.axis_index("c")
        base = wid * ROWS_PER_W
        pltpu.sync_copy(packed_hbm.at[pl.ds(base, ROWS_PER_W)], words_v)

        iota = lax.iota(jnp.int32, 16)
        ones = jnp.ones((16,), jnp.float32)
        zeros = jnp.zeros((16,), jnp.float32)

        @plsc.parallel_loop(0, ROWS_PER_W * VPAD, step=16, unroll=8)
        def _zero(i):
            counts_v[pl.ds(i, 16)] = zeros

        def group_body(g, carry):
            rows = g * 16 + iota
            lanebase = rows * VPAD

            @plsc.parallel_loop(0, NW, unroll=5)
            def _word(p):
                w = plsc.load_gather(words_v, [rows, jnp.full((16,), p, jnp.int32)])
                for k in range(8):
                    v = lax.shift_right_logical(w, 4 * k) & 15 if k else w & 15
                    plsc.addupdate_scatter(counts_v, [lanebase + v], ones)

            return carry

        lax.fori_loop(0, GROUPS, group_body, 0)
        pltpu.sync_copy(
            counts_v, counts_hbm.at[pl.ds(base * VPAD, ROWS_PER_W * VPAD)]
        )

    return hist(packed)


def _tc_matmul(counts, table_pad):
    BM = 2048

    def mm(counts_ref, table_ref, out_ref):
        out_ref[...] = jnp.dot(
            counts_ref[...], table_ref[...], preferred_element_type=jnp.float32
        )

    return pl.pallas_call(
        mm,
        grid=(B // BM,),
        in_specs=[
            pl.BlockSpec((BM, VPAD), lambda i: (i, 0)),
            pl.BlockSpec((VPAD, EMB), lambda i: (0, 0)),
        ],
        out_specs=pl.BlockSpec((BM, EMB), lambda i: (i, 0)),
        out_shape=jax.ShapeDtypeStruct((B, EMB), jnp.float32),
    )(counts, table_pad)


def kernel(sketchs, sketch_lengths, table):
    sketchs = jnp.asarray(sketchs, jnp.int32)
    lengths_col = jnp.asarray(sketch_lengths, jnp.int32).reshape(B, 1)
    s_lo, s_hi = _shift_matrices()
    table_pad = jnp.zeros((VPAD, EMB), jnp.float32).at[:10, :].set(table)
    packed = _tc_pack(sketchs, lengths_col, s_lo, s_hi)
    counts = _sc_histogram(packed).reshape(B, VPAD)
    return _tc_matmul(counts, table_pad)


# speedup vs baseline: 124.5374x; 1.0113x over previous
import functools

import jax
import jax.numpy as jnp
import numpy as np
from jax import lax
from jax.experimental import pallas as pl
from jax.experimental.pallas import tpu as pltpu
from jax.experimental.pallas import tpu_sc as plsc

B = 16384
L = 200
EMB = 128
VPAD = 16
NW = L // 8
NWPAD = 32

NUM_WORKERS = 32
ROWS_PER_W = B // NUM_WORKERS
GROUPS = ROWS_PER_W // 16


def _shift_matrices():
    s_lo = np.zeros((L, NWPAD), np.float32)
    s_hi = np.zeros((L, NWPAD), np.float32)
    for l in range(L):
        p, k = divmod(l, 8)
        if k < 4:
            s_lo[l, p] = float(16 ** k)
        else:
            s_hi[l, p] = float(16 ** (k - 4))
    return jnp.asarray(s_lo, jnp.bfloat16), jnp.asarray(s_hi, jnp.bfloat16)


def _tc_pack(sketchs, lengths_col, s_lo, s_hi):
    BM = 2048

    def pack(x_ref, len_ref, slo_ref, shi_ref, out_ref):
        pos = lax.broadcasted_iota(jnp.int32, (BM, L), 1)
        valid = pos < len_ref[...]
        xm = jnp.where(valid, x_ref[...], 15).astype(jnp.bfloat16)
        y_lo = jnp.dot(xm, slo_ref[...], preferred_element_type=jnp.float32)
        y_hi = jnp.dot(xm, shi_ref[...], preferred_element_type=jnp.float32)
        out_ref[...] = (y_lo + 0.5).astype(jnp.int32) + (
            (y_hi + 0.5).astype(jnp.int32) << 16
        )

    return pl.pallas_call(
        pack,
        grid=(B // BM,),
        in_specs=[
            pl.BlockSpec((BM, L), lambda i: (i, 0)),
            pl.BlockSpec((BM, 1), lambda i: (i, 0)),
            pl.BlockSpec((L, NWPAD), lambda i: (0, 0)),
            pl.BlockSpec((L, NWPAD), lambda i: (0, 0)),
        ],
        out_specs=pl.BlockSpec((BM, NWPAD), lambda i: (i, 0)),
        out_shape=jax.ShapeDtypeStruct((B, NWPAD), jnp.int32),
    )(sketchs, lengths_col, s_lo, s_hi)


def _sc_histogram(packed):
    mesh = plsc.VectorSubcoreMesh(core_axis_name="c", subcore_axis_name="s")

    @functools.partial(
        pl.kernel,
        mesh=mesh,
        out_type=jax.ShapeDtypeStruct((B * VPAD,), jnp.float32),
        compiler_params=pltpu.CompilerParams(
            use_tc_tiling_on_sc=False, needs_layout_passes=False
        ),
        scratch_types=[
            pltpu.VMEM((ROWS_PER_W, NWPAD), jnp.int32),
            pltpu.VMEM((ROWS_PER_W * VPAD * 4,), jnp.float32),
            pltpu.VMEM((ROWS_PER_W * VPAD,), jnp.float32),
        ],
    )
    def hist(packed_hbm, counts_hbm, words_v, banks_v, counts_v):
        wid = lax.axis_index("s") * 2 + lax.axis_index("c")
        base = wid * ROWS_PER_W
        pltpu.sync_copy(packed_hbm.at[pl.ds(base, ROWS_PER_W)], words_v)

        iota = lax.iota(jnp.int32, 16)
        ones = jnp.ones((16,), jnp.float32)
        zeros = jnp.zeros((16,), jnp.float32)

        @plsc.parallel_loop(0, ROWS_PER_W * VPAD * 4, step=16, unroll=8)
        def _zero(i):
            banks_v[pl.ds(i, 16)] = zeros

        def group_body(g, carry):
            rows = g * 16 + iota
            bankbase = rows * (VPAD * 4)

            @plsc.parallel_loop(0, NW, unroll=4)
            def _word(p):
                w = plsc.load_gather(words_v, [rows, jnp.full((16,), p, jnp.int32)])
                for k in range(8):
                    v = lax.shift_right_logical(w, 4 * k) & 15 if k else w & 15
                    plsc.addupdate_scatter(
                        banks_v, [bankbase + ((k % 4) * VPAD) + v], ones
                    )

            return carry

        lax.fori_loop(0, GROUPS, group_body, 0)

        @plsc.parallel_loop(0, ROWS_PER_W, unroll=4)
        def _merge(r):
            b = r * (VPAD * 4)
            c = (
                banks_v[pl.ds(b, 16)]
                + banks_v[pl.ds(b + VPAD, 16)]
                + banks_v[pl.ds(b + 2 * VPAD, 16)]
                + banks_v[pl.ds(b + 3 * VPAD, 16)]
            )
            counts_v[pl.ds(r * VPAD, 16)] = c

        pltpu.sync_copy(
            counts_v, counts_hbm.at[pl.ds(base * VPAD, ROWS_PER_W * VPAD)]
        )

    return hist(packed)


def _tc_matmul(counts, table_pad):
    BM = 2048

    def mm(counts_ref, table_ref, out_ref):
        out_ref[...] = jnp.dot(
            counts_ref[...], table_ref[...], preferred_element_type=jnp.float32
        )

    return pl.pallas_call(
        mm,
        grid=(B // BM,),
        in_specs=[
            pl.BlockSpec((BM, VPAD), lambda i: (i, 0)),
            pl.BlockSpec((VPAD, EMB), lambda i: (0, 0)),
        ],
        out_specs=pl.BlockSpec((BM, EMB), lambda i: (i, 0)),
        out_shape=jax.ShapeDtypeStruct((B, EMB), jnp.float32),
    )(counts, table_pad)


def kernel(sketchs, sketch_lengths, table):
    sketchs = jnp.asarray(sketchs, jnp.int32)
    lengths_col = jnp.asarray(sketch_lengths, jnp.int32).reshape(B, 1)
    s_lo, s_hi = _shift_matrices()
    table_pad = jnp.zeros((VPAD, EMB), jnp.float32).at[:10, :].set(table)
    packed = _tc_pack(sketchs, lengths_col, s_lo, s_hi)
    counts = _sc_histogram(packed).reshape(B, VPAD)
    return _tc_matmul(counts, table_pad)


# speedup vs baseline: 279.0821x; 2.2409x over previous
import functools

import jax
import jax.numpy as jnp
from jax import lax
from jax.experimental import pallas as pl
from jax.experimental.pallas import tpu as pltpu
from jax.experimental.pallas import tpu_sc as plsc

B = 16384
L = 200
EMB = 128
VPAD = 16

NUM_WORKERS = 32
ROWS_PER_W = B // NUM_WORKERS
GROUPS = ROWS_PER_W // 16


def _sc_histogram_t(sk_t, lengths):
    mesh = plsc.VectorSubcoreMesh(core_axis_name="c", subcore_axis_name="s")

    @functools.partial(
        pl.kernel,
        mesh=mesh,
        out_type=jax.ShapeDtypeStruct((B * VPAD,), jnp.float32),
        compiler_params=pltpu.CompilerParams(
            use_tc_tiling_on_sc=True, needs_layout_passes=False
        ),
        scratch_types=[
            pltpu.VMEM((L, ROWS_PER_W), jnp.int32),
            pltpu.VMEM((ROWS_PER_W,), jnp.int32),
            pltpu.VMEM((ROWS_PER_W * VPAD,), jnp.float32),
        ],
    )
    def hist(skt_hbm, len_hbm, counts_hbm, slab_v, lens_v, counts_v):
        wid = lax.axis_index("s") * 2 + lax.axis_index("c")
        base = wid * ROWS_PER_W
        pltpu.sync_copy(skt_hbm.at[:, pl.ds(base, ROWS_PER_W)], slab_v)
        pltpu.sync_copy(len_hbm.at[pl.ds(base, ROWS_PER_W)], lens_v)

        iota = lax.iota(jnp.int32, 16)
        ones = jnp.ones((16,), jnp.float32)
        zeros = jnp.zeros((16,), jnp.float32)

        @plsc.parallel_loop(0, ROWS_PER_W * VPAD, step=16, unroll=8)
        def _zero(i):
            counts_v[pl.ds(i, 16)] = zeros

        def group_body(g, carry):
            lanebase = (g * 16 + iota) * VPAD
            lens16 = lens_v[pl.ds(g * 16, 16)]

            @plsc.parallel_loop(0, L, unroll=8)
            def _pos(l):
                vals = slab_v[l, pl.ds(g * 16, 16)]
                mask = l < lens16
                plsc.addupdate_scatter(counts_v, [lanebase + vals], ones, mask=mask)

            return carry

        lax.fori_loop(0, GROUPS, group_body, 0)
        pltpu.sync_copy(counts_v, counts_hbm.at[pl.ds(base * VPAD, ROWS_PER_W * VPAD)])

    return hist(sk_t, lengths)


def _tc_matmul(counts, table_pad):
    BM = 2048

    def mm(counts_ref, table_ref, out_ref):
        out_ref[...] = jnp.dot(
            counts_ref[...], table_ref[...], preferred_element_type=jnp.float32
        )

    return pl.pallas_call(
        mm,
        grid=(B // BM,),
        in_specs=[
            pl.BlockSpec((BM, VPAD), lambda i: (i, 0)),
            pl.BlockSpec((VPAD, EMB), lambda i: (0, 0)),
        ],
        out_specs=pl.BlockSpec((BM, EMB), lambda i: (i, 0)),
        out_shape=jax.ShapeDtypeStruct((B, EMB), jnp.float32),
    )(counts, table_pad)


def kernel(sketchs, sketch_lengths, table):
    sk_t = jnp.transpose(jnp.asarray(sketchs, jnp.int32))
    lengths = jnp.asarray(sketch_lengths, jnp.int32)
    table_pad = jnp.zeros((VPAD, EMB), jnp.float32).at[:10, :].set(table)
    counts = _sc_histogram_t(sk_t, lengths).reshape(B, VPAD)
    return _tc_matmul(counts, table_pad)


# speedup vs baseline: 340.9041x; 1.2215x over previous
import functools

import jax
import jax.numpy as jnp
from jax import lax
from jax.experimental import pallas as pl
from jax.experimental.pallas import tpu as pltpu
from jax.experimental.pallas import tpu_sc as plsc

B = 16384
L = 200
EMB = 128
VPAD = 16

NUM_WORKERS = 32
ROWS_PER_W = B // NUM_WORKERS
GROUPS = ROWS_PER_W // 16


def _sc_histogram_t(sk_t, lengths):
    mesh = plsc.VectorSubcoreMesh(core_axis_name="c", subcore_axis_name="s")

    @functools.partial(
        pl.kernel,
        mesh=mesh,
        out_type=jax.ShapeDtypeStruct((VPAD, B), jnp.float32),
        compiler_params=pltpu.CompilerParams(
            use_tc_tiling_on_sc=True, needs_layout_passes=False
        ),
        scratch_types=[
            pltpu.VMEM((L, ROWS_PER_W), jnp.int32),
            pltpu.VMEM((ROWS_PER_W,), jnp.int32),
            pltpu.VMEM((VPAD, ROWS_PER_W), jnp.float32),
        ],
    )
    def hist(skt_hbm, len_hbm, counts_hbm, slab_v, lens_v, counts_v):
        wid = lax.axis_index("s") * 2 + lax.axis_index("c")
        base = wid * ROWS_PER_W
        pltpu.sync_copy(skt_hbm.at[:, pl.ds(base, ROWS_PER_W)], slab_v)
        pltpu.sync_copy(len_hbm.at[pl.ds(base, ROWS_PER_W)], lens_v)

        iota = lax.iota(jnp.int32, 16)
        ones = jnp.ones((16,), jnp.float32)
        zeros = jnp.zeros((16,), jnp.float32)

        def zero_row(v, carry):
            @plsc.parallel_loop(0, ROWS_PER_W, step=16, unroll=8)
            def _zero(i):
                counts_v[v, pl.ds(i, 16)] = zeros

            return carry

        lax.fori_loop(0, VPAD, zero_row, 0)

        def group_body(g, carry):
            rows = g * 16 + iota
            lens16 = lens_v[pl.ds(g * 16, 16)]

            @plsc.parallel_loop(0, L, unroll=8)
            def _pos(l):
                vals = slab_v[l, pl.ds(g * 16, 16)]
                mask = l < lens16
                plsc.addupdate_scatter(counts_v, [vals, rows], ones, mask=mask)

            return carry

        lax.fori_loop(0, GROUPS, group_body, 0)
        pltpu.sync_copy(counts_v, counts_hbm.at[:, pl.ds(base, ROWS_PER_W)])

    return hist(sk_t, lengths)


def _tc_matmul(counts_t, table_pad):
    BM = 4096

    def mm(counts_ref, table_ref, out_ref):
        out_ref[...] = lax.dot_general(
            counts_ref[...],
            table_ref[...],
            (((0,), (0,)), ((), ())),
            preferred_element_type=jnp.float32,
        )

    return pl.pallas_call(
        mm,
        grid=(B // BM,),
        in_specs=[
            pl.BlockSpec((VPAD, BM), lambda i: (0, i)),
            pl.BlockSpec((VPAD, EMB), lambda i: (0, 0)),
        ],
        out_specs=pl.BlockSpec((BM, EMB), lambda i: (i, 0)),
        out_shape=jax.ShapeDtypeStruct((B, EMB), jnp.float32),
    )(counts_t, table_pad)


def kernel(sketchs, sketch_lengths, table):
    sk_t = jnp.transpose(jnp.asarray(sketchs, jnp.int32))
    lengths = jnp.asarray(sketch_lengths, jnp.int32)
    table_pad = jnp.zeros((VPAD, EMB), jnp.float32).at[:10, :].set(table)
    counts_t = _sc_histogram_t(sk_t, lengths)
    return _tc_matmul(counts_t, table_pad)
